# baseline (speedup 1.0000x reference)
import functools

import jax
import jax.numpy as jnp
from jax import lax
from jax.experimental import pallas as pl
from jax.experimental.pallas import tpu as pltpu
from jax.experimental.pallas import tpu_sc as plsc

VOCAB = 1_000_000
D = 64
BATCH = 4096
SEQ = 200
SCALE = 8.0

NC, NS, L = 2, 16, 16
NW = NC * NS
BB = BATCH // NW
_MESH = plsc.VectorSubcoreMesh(
    core_axis_name="c", subcore_axis_name="s",
    num_cores=NC, num_subcores=NS)


@functools.partial(
    pl.kernel,
    out_type=jax.ShapeDtypeStruct((SEQ * 8, NW, 8 * BB), jnp.float32),
    mesh=_MESH,
    scratch_types=[
        pltpu.VMEM((SEQ // 4, 4 * BB), jnp.int32),
        pltpu.VMEM((4 * BB, D), jnp.float32),
        pltpu.VMEM((4 * BB, D), jnp.float32),
        pltpu.VMEM((8, 8 * BB), jnp.float32),
        pltpu.VMEM((8, 8 * BB), jnp.float32),
        pltpu.SemaphoreType.DMA,
        pltpu.SemaphoreType.DMA,
        pltpu.SemaphoreType.DMA,
        pltpu.SemaphoreType.DMA,
    ],
    compiler_params=pltpu.CompilerParams(
        use_tc_tiling_on_sc=False, needs_layout_passes=False),
)
def _embed_sc(tokshuf_hbm, tlin_hbm, out_hbm, idxv, rows0, rows1,
              st0, st1, sg0, sg1, so0, so1):
    wid = lax.axis_index("s") * NC + lax.axis_index("c")
    lanes = lax.iota(jnp.int32, L)
    NGB = SEQ // 4

    pltpu.sync_copy(tokshuf_hbm.at[wid], idxv)

    def _gather(g, rows, sem):
        pltpu.async_copy(tlin_hbm.at[idxv.at[g]], rows, sem)

    def _tpose(rows, base, st):
        for b0 in range(BB // L):
            bv = lanes + (base + b0 * L)

            @plsc.parallel_loop(0, D, unroll=16)
            def _feat(c, rows=rows, st=st, bv=bv, b0=b0):
                cv = jnp.zeros((L,), jnp.int32) + c
                v = plsc.load_gather(rows, [bv, cv])
                st[c // 8, pl.ds((c % 8) * BB + b0 * L, L)] = v * SCALE

    def _fire_out(s, st, so):
        pltpu.async_copy(
            st, out_hbm.at[pl.ds(pl.multiple_of(s * 8, 8), 8), wid], so)

    def _drain_out(st, so):
        pltpu.make_async_copy(st, out_hbm.at[pl.ds(0, 8), 0], so).wait()

    _gather(0, rows0, sg0)
    _gather(1, rows1, sg1)

    @pl.loop(0, NGB // 2)
    def _pos(gg):
        for p, (rows, sg) in enumerate(((rows0, sg0), (rows1, sg1))):
            g = 2 * gg + p
            pltpu.make_async_copy(tlin_hbm.at[idxv.at[0]], rows, sg).wait()
            for sl in range(4):
                st, so = (st0, so0) if sl % 2 == 0 else (st1, so1)
                s = g * 4 + sl

                @pl.when(s >= 2)
                def _(st=st, so=so):
                    _drain_out(st, so)

                _tpose(rows, sl * BB, st)
                _fire_out(s, st, so)

            @pl.when(g < NGB - 2)
            def _(g=g, rows=rows, sg=sg):
                _gather(g + 2, rows, sg)

    _drain_out(st0, so0)
    _drain_out(st1, so1)


def kernel(tokens, table):
    tokshuf = (tokens.astype(jnp.int32).T
               .reshape(SEQ // 4, 4, NW, BB)
               .transpose(2, 0, 1, 3)
               .reshape(NW, SEQ // 4, 4 * BB))
    res5 = _embed_sc(tokshuf, table).reshape(SEQ, 8, NW, 8, BB)
    return res5.transpose(2, 4, 0, 1, 3).reshape(BATCH, SEQ, D)
